# Initial kernel scaffold; baseline (speedup 1.0000x reference)
import jax
import jax.numpy as jnp
from jax.experimental import pallas as pl


def kernel(entities, dt, abst, ent_embs, w, b, t_w, abst_embs):
    raise NotImplementedError("write your pallas kernel here")


# speedup vs baseline: 3.2123x; 3.2123x over previous
import jax
import jax.numpy as jnp
from jax import lax
from jax.experimental import pallas as pl
from jax.experimental.pallas import tpu as pltpu
from jax.experimental.pallas import tpu_sc as plsc

B, S = 4096, 50
BS = B * S
N_ENT, N_ABST = 100000, 365
DIM_E = 96
DIM_T = 32
DIM_O = 128

NC, NS, L = 2, 16, 16
NW = NC * NS
PER_W = BS // NW
CHUNK = 256
N_CHUNK = PER_W // CHUNK

_TWO_PI = 6.283185307179586
_PI = 3.141592653589793
_C = (0.9999999848280942, -0.4999998592246148, 0.04166645028442878,
      -0.0013887630148866539, 2.4766406142774644e-05,
      -2.704700886591027e-07, 1.7134819119135614e-09)


def _cos_vec(x):
    y = lax.rem(x, jnp.float32(_TWO_PI))
    y = jnp.where(y < 0.0, y + jnp.float32(_TWO_PI), y)
    z = y - jnp.float32(_PI)
    u = z * z
    p = jnp.full_like(u, _C[6])
    for k in (5, 4, 3, 2, 1, 0):
        p = p * u + jnp.float32(_C[k])
    return -p


def _sc_kernel(ent_hbm, dt_hbm, abst_hbm, tab_hbm, w_hbm, b_hbm, tw_hbm,
               atab_hbm, out_hbm,
               idx_v, day_v, dt_v, atab_v, outb,
               w_v, b_v, tw_v, sem_e):
    wid = lax.axis_index("s") * NC + lax.axis_index("c")
    base = wid * PER_W

    pltpu.sync_copy(w_hbm, w_v)
    pltpu.sync_copy(b_hbm, b_v)
    pltpu.sync_copy(tw_hbm, tw_v)
    pltpu.sync_copy(atab_hbm, atab_v)

    halves = []
    for h in range(2):
        sl = pl.ds(h * L, L)
        wv, bv, twv = w_v[sl], b_v[sl], tw_v[sl]
        s = 1.0 / (1.0 + jnp.exp(-twv))
        halves.append((wv, bv, s, s - 1.0))

    def chunk_body(c, _):
        off = base + c * CHUNK
        pltpu.sync_copy(ent_hbm.at[pl.ds(off, CHUNK)], idx_v)
        cp_e = pltpu.make_async_copy(tab_hbm.at[idx_v], outb, sem_e)
        cp_e.start()
        pltpu.sync_copy(abst_hbm.at[pl.ds(off, CHUNK)], day_v)
        for g in range(CHUNK // L):
            sl = pl.ds(g * L, L)
            day_v[sl] = lax.div(day_v[sl], jnp.int32(24))
        pltpu.sync_copy(dt_hbm.at[pl.ds(off, CHUNK)], dt_v)
        cp_e.wait()

        def grp_body(g, _):
            dtv = dt_v[pl.ds(g * L, L)]
            dayv = day_v[pl.ds(g * L, L)]
            for l in range(L):
                p = g * L + l
                dtp = dtv[l]
                dayp = dayv[l]
                for h, (wv, bv, s, sm1) in enumerate(halves):
                    x = wv * dtp + bv
                    cz = _cos_vec(x)
                    a = atab_v[dayp, pl.ds(h * L, L)]
                    outb[p, pl.ds(DIM_E + h * L, L)] = s * a - sm1 * cz
            return 0

        lax.fori_loop(0, CHUNK // L, grp_body, 0)
        pltpu.sync_copy(outb, out_hbm.at[pl.ds(off, CHUNK)])
        return 0

    lax.fori_loop(0, N_CHUNK, chunk_body, 0)


@jax.jit
def kernel(entities, dt, abst, ent_embs, w, b, t_w, abst_embs):
    ent_flat = entities.reshape(BS).astype(jnp.int32)
    abst_flat = abst.reshape(BS).astype(jnp.int32)
    dt_flat = dt.reshape(BS).astype(jnp.float32)
    tab_pad = jnp.pad(ent_embs, ((0, 0), (0, DIM_O - DIM_E)))
    atab = abst_embs

    mesh = plsc.VectorSubcoreMesh(core_axis_name="c", subcore_axis_name="s")
    run = pl.kernel(
        _sc_kernel,
        out_type=jax.ShapeDtypeStruct((BS, DIM_O), jnp.float32),
        mesh=mesh,
        scratch_types=[
            pltpu.VMEM((CHUNK,), jnp.int32),
            pltpu.VMEM((CHUNK,), jnp.int32),
            pltpu.VMEM((CHUNK,), jnp.float32),
            pltpu.VMEM((N_ABST, DIM_T), jnp.float32),
            pltpu.VMEM((CHUNK, DIM_O), jnp.float32),
            pltpu.VMEM((DIM_T,), jnp.float32),
            pltpu.VMEM((DIM_T,), jnp.float32),
            pltpu.VMEM((DIM_T,), jnp.float32),
            pltpu.SemaphoreType.DMA,
        ],
    )
    out = run(ent_flat, dt_flat, abst_flat, tab_pad, w, b, t_w, atab)
    return out.reshape(B, S, DIM_O)


# speedup vs baseline: 5.7465x; 1.7889x over previous
import jax
import jax.numpy as jnp
from jax import lax
from jax.experimental import pallas as pl
from jax.experimental.pallas import tpu as pltpu
from jax.experimental.pallas import tpu_sc as plsc

B, S = 4096, 50
BS = B * S
N_ENT, N_ABST = 100000, 365
DIM_E = 96
DIM_T = 32
DIM_O = 128

NC, NS, L = 2, 16, 16
NW = NC * NS
PER_W = BS // NW
CHUNK = 160
N_CHUNK = PER_W // CHUNK
PACK = 4

_PAD_BLK = 2048


def _pad_tc_kernel(src_ref, out_ref):
    x = src_ref[...]
    y = jnp.swapaxes(x, 0, 1)
    out_ref[...] = jnp.concatenate(
        [y, jnp.zeros((_PAD_BLK, DIM_O - DIM_E), jnp.float32)], axis=1)


def _pad_table(ent_embs):
    n_blk = (N_ENT + _PAD_BLK - 1) // _PAD_BLK
    return pl.pallas_call(
        _pad_tc_kernel,
        grid=(n_blk,),
        in_specs=[pl.BlockSpec((DIM_E, _PAD_BLK), lambda i: (0, i))],
        out_specs=pl.BlockSpec((_PAD_BLK, DIM_O), lambda i: (i, 0)),
        out_shape=jax.ShapeDtypeStruct((n_blk * _PAD_BLK, DIM_O), jnp.float32),
    )(ent_embs.T)


_COS_BLK = 512


def _cos_tc_kernel(dt4_ref, w_ref, b_ref, oms_ref, out_ref):
    xb = dt4_ref[...]
    cols = [jnp.broadcast_to(xb[:, j:j + 1], (_COS_BLK, DIM_T))
            for j in range(PACK)]
    dtrep = jnp.concatenate(cols, axis=1)
    out_ref[...] = oms_ref[...] * jnp.cos(w_ref[...] * dtrep + b_ref[...])


def _cos_packed(dt4, w128, b128, oms128):
    n_rows = BS // PACK
    return pl.pallas_call(
        _cos_tc_kernel,
        grid=(n_rows // _COS_BLK,),
        in_specs=[
            pl.BlockSpec((_COS_BLK, PACK), lambda i: (i, 0)),
            pl.BlockSpec((DIM_O,), lambda i: (0,)),
            pl.BlockSpec((DIM_O,), lambda i: (0,)),
            pl.BlockSpec((DIM_O,), lambda i: (0,)),
        ],
        out_specs=pl.BlockSpec((_COS_BLK, DIM_O), lambda i: (i, 0)),
        out_shape=jax.ShapeDtypeStruct((n_rows, DIM_O), jnp.float32),
    )(dt4, w128, b128, oms128)


def _sc_kernel(ent_hbm, abst_hbm, tc_hbm, tab_hbm, tw_hbm, atab_hbm,
               out_hbm,
               idx_all, day_all, atab_v, tw_v,
               tc0, tc1, outb0, outb1,
               se0, se1, st0, st1, sw0, sw1):
    wid = lax.axis_index("s") * NC + lax.axis_index("c")
    base = pl.multiple_of(wid * PER_W, PER_W)
    base4 = pl.multiple_of(wid * (PER_W // PACK), PER_W // PACK)

    pltpu.sync_copy(tw_hbm, tw_v)
    pltpu.sync_copy(atab_hbm, atab_v)
    pltpu.sync_copy(ent_hbm.at[pl.ds(base, PER_W)], idx_all)
    pltpu.sync_copy(abst_hbm.at[pl.ds(base, PER_W)], day_all)

    s_h = []
    for h in range(2):
        twv = tw_v[pl.ds(h * L, L)]
        s_h.append(1.0 / (1.0 + jnp.exp(-twv)))

    tc_b = (tc0, tc1)
    outb_b = (outb0, outb1)
    se_b = (se0, se1)
    st_b = (st0, st1)
    sw_b = (sw0, sw1)

    def chunk_copies(c, d):
        lo = pl.multiple_of(c * CHUNK, CHUNK)
        lo4 = pl.multiple_of(c * (CHUNK // PACK), CHUNK // PACK)
        cp_e = pltpu.make_async_copy(
            tab_hbm.at[idx_all.at[pl.ds(lo, CHUNK)]], outb_b[d], se_b[d])
        cp_t = pltpu.make_async_copy(
            tc_hbm.at[pl.ds(pl.multiple_of(base4 + lo4, 8), CHUNK // PACK)],
            tc_b[d], st_b[d])
        return cp_e, cp_t

    def start_chunk(c, d):
        cp_e, cp_t = chunk_copies(c, d)
        cp_e.start()
        cp_t.start()

    def wait_chunk(c, d):
        cp_e, cp_t = chunk_copies(c, d)
        cp_e.wait()
        cp_t.wait()

    def write_copy(c, d):
        lo = pl.multiple_of(c * CHUNK, CHUNK)
        return pltpu.make_async_copy(
            outb_b[d],
            out_hbm.at[pl.ds(pl.multiple_of(base + lo, CHUNK), CHUNK)],
            sw_b[d])

    def compute_chunk(c, d):
        outb, tcv_buf = outb_b[d], tc_b[d]
        lo = pl.multiple_of(c * CHUNK, CHUNK)

        def grp_body(g, _):
            goff = pl.multiple_of(lo + g * L, L)
            dayv = lax.div(day_all[pl.ds(goff, L)], jnp.int32(24))
            for l in range(L):
                p = g * L + l
                dayp = dayv[l]
                row = g * (L // PACK) + l // PACK
                col = DIM_T * (l % PACK)
                for h in range(2):
                    a = atab_v[dayp, pl.ds(h * L, L)]
                    t = tcv_buf[row, pl.ds(col + h * L, L)]
                    outb[p, pl.ds(DIM_E + h * L, L)] = t + s_h[h] * a
            return 0

        lax.fori_loop(0, CHUNK // L, grp_body, 0)
        write_copy(c, d).start()

    start_chunk(0, 0)

    wait_chunk(0, 0)
    start_chunk(1, 1)
    compute_chunk(0, 0)

    def pair_body(k, _):
        c1 = 2 * k + 1
        wait_chunk(c1, 1)
        write_copy(c1 - 1, 0).wait()
        start_chunk(c1 + 1, 0)
        compute_chunk(c1, 1)

        c2 = 2 * k + 2
        wait_chunk(c2, 0)
        write_copy(c2 - 1, 1).wait()
        start_chunk(c2 + 1, 1)
        compute_chunk(c2, 0)
        return 0

    lax.fori_loop(0, (N_CHUNK - 2) // 2, pair_body, 0)

    wait_chunk(N_CHUNK - 1, 1)
    write_copy(N_CHUNK - 2, 0).wait()
    compute_chunk(N_CHUNK - 1, 1)
    write_copy(N_CHUNK - 1, 1).wait()


@jax.jit
def kernel(entities, dt, abst, ent_embs, w, b, t_w, abst_embs):
    ent_flat = entities.T.reshape(BS).astype(jnp.int32)
    abst_flat = abst.T.reshape(BS).astype(jnp.int32)
    dt4 = dt.T.astype(jnp.float32).reshape(BS // PACK, PACK)

    w128 = jnp.tile(w.astype(jnp.float32), PACK)
    b128 = jnp.tile(b.astype(jnp.float32), PACK)
    oms128 = jnp.tile(1.0 - jax.nn.sigmoid(t_w.astype(jnp.float32)), PACK)

    tab_pad = _pad_table(ent_embs)
    tc_packed = _cos_packed(dt4, w128, b128, oms128)

    mesh = plsc.VectorSubcoreMesh(core_axis_name="c", subcore_axis_name="s")
    run = pl.kernel(
        _sc_kernel,
        out_type=jax.ShapeDtypeStruct((BS, DIM_O), jnp.float32),
        mesh=mesh,
        scratch_types=[
            pltpu.VMEM((PER_W,), jnp.int32),
            pltpu.VMEM((PER_W,), jnp.int32),
            pltpu.VMEM((N_ABST, DIM_T), jnp.float32),
            pltpu.VMEM((DIM_T,), jnp.float32),
            pltpu.VMEM((CHUNK // PACK, DIM_O), jnp.float32),
            pltpu.VMEM((CHUNK // PACK, DIM_O), jnp.float32),
            pltpu.VMEM((CHUNK, DIM_O), jnp.float32),
            pltpu.VMEM((CHUNK, DIM_O), jnp.float32),
            pltpu.SemaphoreType.DMA,
            pltpu.SemaphoreType.DMA,
            pltpu.SemaphoreType.DMA,
            pltpu.SemaphoreType.DMA,
            pltpu.SemaphoreType.DMA,
            pltpu.SemaphoreType.DMA,
        ],
    )
    out = run(ent_flat, abst_flat, tc_packed, tab_pad, t_w, abst_embs)
    return out.reshape(S, B, DIM_O).transpose(1, 0, 2)


# speedup vs baseline: 7.0759x; 1.2313x over previous
import jax
import jax.numpy as jnp
from jax import lax
from jax.experimental import pallas as pl
from jax.experimental.pallas import tpu as pltpu
from jax.experimental.pallas import tpu_sc as plsc

B, S = 4096, 50
BS = B * S
N_ENT, N_ABST = 100000, 365
DIM_E = 96
DIM_T = 32
DIM_O = 128

NC, NS, L = 2, 16, 16
NW = NC * NS
PER_W = BS // NW
CHUNK = 160
N_CHUNK = PER_W // CHUNK
PACK = 4

_PAD_BLK = 2048


def _pad_tc_kernel(src_ref, out_ref):
    x = src_ref[...]
    y = jnp.swapaxes(x, 0, 1)
    out_ref[...] = jnp.concatenate(
        [y, jnp.zeros((_PAD_BLK, DIM_O - DIM_E), jnp.float32)], axis=1)


def _pad_table(ent_embs):
    n_blk = (N_ENT + _PAD_BLK - 1) // _PAD_BLK
    return pl.pallas_call(
        _pad_tc_kernel,
        grid=(n_blk,),
        in_specs=[pl.BlockSpec((DIM_E, _PAD_BLK), lambda i: (0, i))],
        out_specs=pl.BlockSpec((_PAD_BLK, DIM_O), lambda i: (i, 0)),
        out_shape=jax.ShapeDtypeStruct((n_blk * _PAD_BLK, DIM_O), jnp.float32),
    )(ent_embs.T)


_COS_BLK = 2048

_TWO_PI = 6.283185307179586
_PI = 3.141592653589793
_INV_TWO_PI = 0.15915494309189535
_C = (0.9999999848280942, -0.4999998592246148, 0.04166645028442878,
      -0.0013887630148866539, 2.4766406142774644e-05,
      -2.704700886591027e-07, 1.7134819119135614e-09)


def _neg_cos(x):
    k = (x * jnp.float32(_INV_TWO_PI)).astype(jnp.int32)
    y = x - k.astype(jnp.float32) * jnp.float32(_TWO_PI)
    y = jnp.where(y < 0.0, y + jnp.float32(_TWO_PI), y)
    z = y - jnp.float32(_PI)
    u = z * z
    p = jnp.full_like(u, _C[6])
    for i in (5, 4, 3, 2, 1, 0):
        p = p * u + jnp.float32(_C[i])
    return p


def _cos_tc_kernel(dt4_ref, w_ref, b_ref, oms_ref, out_ref):
    xb = dt4_ref[...]
    cols = [jnp.broadcast_to(xb[:, j:j + 1], (_COS_BLK, DIM_T))
            for j in range(PACK)]
    dtrep = jnp.concatenate(cols, axis=1)
    out_ref[...] = (-oms_ref[...]) * _neg_cos(w_ref[...] * dtrep + b_ref[...])


def _cos_packed(dt4, w128, b128, oms128):
    n_rows = BS // PACK
    return pl.pallas_call(
        _cos_tc_kernel,
        grid=(n_rows // _COS_BLK,),
        in_specs=[
            pl.BlockSpec((_COS_BLK, PACK), lambda i: (i, 0)),
            pl.BlockSpec((DIM_O,), lambda i: (0,)),
            pl.BlockSpec((DIM_O,), lambda i: (0,)),
            pl.BlockSpec((DIM_O,), lambda i: (0,)),
        ],
        out_specs=pl.BlockSpec((_COS_BLK, DIM_O), lambda i: (i, 0)),
        out_shape=jax.ShapeDtypeStruct((n_rows, DIM_O), jnp.float32),
    )(dt4, w128, b128, oms128)


def _sc_kernel(ent_hbm, abst_hbm, tc_hbm, tab_hbm, tw_hbm, atab_hbm,
               out_hbm,
               idx_all, day_all, atab_v, tw_v,
               tc0, tc1, outb0, outb1,
               se0, se1, st0, st1, sw0, sw1):
    wid = lax.axis_index("s") * NC + lax.axis_index("c")
    base = pl.multiple_of(wid * PER_W, PER_W)
    base4 = pl.multiple_of(wid * (PER_W // PACK), PER_W // PACK)

    pltpu.sync_copy(tw_hbm, tw_v)
    pltpu.sync_copy(atab_hbm, atab_v)
    pltpu.sync_copy(ent_hbm.at[pl.ds(base, PER_W)], idx_all)
    pltpu.sync_copy(abst_hbm.at[pl.ds(base, PER_W)], day_all)

    s_h = []
    for h in range(2):
        twv = tw_v[pl.ds(h * L, L)]
        s_h.append(1.0 / (1.0 + jnp.exp(-twv)))

    tc_b = (tc0, tc1)
    outb_b = (outb0, outb1)
    se_b = (se0, se1)
    st_b = (st0, st1)
    sw_b = (sw0, sw1)

    def chunk_copies(c, d):
        lo = pl.multiple_of(c * CHUNK, CHUNK)
        lo4 = pl.multiple_of(c * (CHUNK // PACK), CHUNK // PACK)
        cp_e = pltpu.make_async_copy(
            tab_hbm.at[idx_all.at[pl.ds(lo, CHUNK)]], outb_b[d], se_b[d])
        cp_t = pltpu.make_async_copy(
            tc_hbm.at[pl.ds(pl.multiple_of(base4 + lo4, 8), CHUNK // PACK)],
            tc_b[d], st_b[d])
        return cp_e, cp_t

    def start_chunk(c, d):
        cp_e, cp_t = chunk_copies(c, d)
        cp_e.start()
        cp_t.start()

    def wait_chunk(c, d):
        cp_e, cp_t = chunk_copies(c, d)
        cp_e.wait()
        cp_t.wait()

    def write_copy(c, d):
        lo = pl.multiple_of(c * CHUNK, CHUNK)
        return pltpu.make_async_copy(
            outb_b[d],
            out_hbm.at[pl.ds(pl.multiple_of(base + lo, CHUNK), CHUNK)],
            sw_b[d])

    def compute_chunk(c, d):
        outb, tcv_buf = outb_b[d], tc_b[d]
        lo = pl.multiple_of(c * CHUNK, CHUNK)

        def grp_body(g, _):
            goff = pl.multiple_of(lo + g * L, L)
            dayv = lax.div(day_all[pl.ds(goff, L)], jnp.int32(24))
            for l in range(L):
                p = g * L + l
                dayp = dayv[l]
                row = g * (L // PACK) + l // PACK
                col = DIM_T * (l % PACK)
                for h in range(2):
                    a = atab_v[dayp, pl.ds(h * L, L)]
                    t = tcv_buf[row, pl.ds(col + h * L, L)]
                    outb[p, pl.ds(DIM_E + h * L, L)] = t + s_h[h] * a
            return 0

        lax.fori_loop(0, CHUNK // L, grp_body, 0)
        write_copy(c, d).start()

    start_chunk(0, 0)

    wait_chunk(0, 0)
    start_chunk(1, 1)
    compute_chunk(0, 0)

    def pair_body(k, _):
        c1 = 2 * k + 1
        wait_chunk(c1, 1)
        write_copy(c1 - 1, 0).wait()
        start_chunk(c1 + 1, 0)
        compute_chunk(c1, 1)

        c2 = 2 * k + 2
        wait_chunk(c2, 0)
        write_copy(c2 - 1, 1).wait()
        start_chunk(c2 + 1, 1)
        compute_chunk(c2, 0)
        return 0

    lax.fori_loop(0, (N_CHUNK - 2) // 2, pair_body, 0)

    wait_chunk(N_CHUNK - 1, 1)
    write_copy(N_CHUNK - 2, 0).wait()
    compute_chunk(N_CHUNK - 1, 1)
    write_copy(N_CHUNK - 1, 1).wait()


@jax.jit
def kernel(entities, dt, abst, ent_embs, w, b, t_w, abst_embs):
    ent_flat = entities.T.reshape(BS).astype(jnp.int32)
    abst_flat = abst.T.reshape(BS).astype(jnp.int32)
    dt4 = dt.T.astype(jnp.float32).reshape(BS // PACK, PACK)

    w128 = jnp.tile(w.astype(jnp.float32), PACK)
    b128 = jnp.tile(b.astype(jnp.float32), PACK)
    oms128 = jnp.tile(1.0 - jax.nn.sigmoid(t_w.astype(jnp.float32)), PACK)

    tab_pad = _pad_table(ent_embs)
    tc_packed = _cos_packed(dt4, w128, b128, oms128)

    mesh = plsc.VectorSubcoreMesh(core_axis_name="c", subcore_axis_name="s")
    run = pl.kernel(
        _sc_kernel,
        out_type=jax.ShapeDtypeStruct((BS, DIM_O), jnp.float32),
        mesh=mesh,
        scratch_types=[
            pltpu.VMEM((PER_W,), jnp.int32),
            pltpu.VMEM((PER_W,), jnp.int32),
            pltpu.VMEM((N_ABST, DIM_T), jnp.float32),
            pltpu.VMEM((DIM_T,), jnp.float32),
            pltpu.VMEM((CHUNK // PACK, DIM_O), jnp.float32),
            pltpu.VMEM((CHUNK // PACK, DIM_O), jnp.float32),
            pltpu.VMEM((CHUNK, DIM_O), jnp.float32),
            pltpu.VMEM((CHUNK, DIM_O), jnp.float32),
            pltpu.SemaphoreType.DMA,
            pltpu.SemaphoreType.DMA,
            pltpu.SemaphoreType.DMA,
            pltpu.SemaphoreType.DMA,
            pltpu.SemaphoreType.DMA,
            pltpu.SemaphoreType.DMA,
        ],
    )
    out = run(ent_flat, abst_flat, tc_packed, tab_pad, t_w, abst_embs)
    return out.reshape(S, B, DIM_O).transpose(1, 0, 2)


# speedup vs baseline: 9.0603x; 1.2804x over previous
import jax
import jax.numpy as jnp
from jax import lax
from jax.experimental import pallas as pl
from jax.experimental.pallas import tpu as pltpu
from jax.experimental.pallas import tpu_sc as plsc

B, S = 4096, 50
BS = B * S
N_ENT, N_ABST = 100000, 365
DIM_E = 96
DIM_T = 32
DIM_O = 128

NC, NS, L = 2, 16, 16
NW = NC * NS
PER_W = BS // NW
CHUNK = 160
N_CHUNK = PER_W // CHUNK

_TWO_PI = 6.283185307179586
_PI = 3.141592653589793
_INV_TWO_PI = 0.15915494309189535
_C = (0.9999999848280942, -0.4999998592246148, 0.04166645028442878,
      -0.0013887630148866539, 2.4766406142774644e-05,
      -2.704700886591027e-07, 1.7134819119135614e-09)


def _neg_cos(x):
    k = (x * jnp.float32(_INV_TWO_PI)).astype(jnp.int32)
    y = x - k.astype(jnp.float32) * jnp.float32(_TWO_PI)
    y = jnp.where(y < 0.0, y + jnp.float32(_TWO_PI), y)
    z = y - jnp.float32(_PI)
    u = z * z
    p = jnp.full_like(u, _C[6])
    for i in (5, 4, 3, 2, 1, 0):
        p = p * u + jnp.float32(_C[i])
    return p


_PAD_BLK = 2048


def _pad_tc_kernel(src_ref, out_ref):
    x = src_ref[...]
    y = jnp.swapaxes(x, 0, 1)
    out_ref[...] = jnp.concatenate(
        [y, jnp.zeros((_PAD_BLK, DIM_O - DIM_E), jnp.float32)], axis=1)


def _pad_table(ent_embs):
    n_blk = (N_ENT + _PAD_BLK - 1) // _PAD_BLK
    return pl.pallas_call(
        _pad_tc_kernel,
        grid=(n_blk,),
        in_specs=[pl.BlockSpec((DIM_E, _PAD_BLK), lambda i: (0, i))],
        out_specs=pl.BlockSpec((_PAD_BLK, DIM_O), lambda i: (i, 0)),
        out_shape=jax.ShapeDtypeStruct((n_blk * _PAD_BLK, DIM_O), jnp.float32),
    )(ent_embs.T)


def _sc_kernel(ent_hbm, abst_hbm, dt_hbm, tab_hbm, w_hbm, b_hbm, tw_hbm,
               atab_hbm, out_hbm,
               idx_all, day_all, dt_all, atab_v, w_v, b_v, tw_v,
               outb0, outb1,
               se0, se1, sw0, sw1):
    wid = lax.axis_index("s") * NC + lax.axis_index("c")
    base = pl.multiple_of(wid * PER_W, PER_W)

    pltpu.sync_copy(w_hbm, w_v)
    pltpu.sync_copy(b_hbm, b_v)
    pltpu.sync_copy(tw_hbm, tw_v)
    pltpu.sync_copy(atab_hbm, atab_v)
    pltpu.sync_copy(ent_hbm.at[pl.ds(base, PER_W)], idx_all)
    pltpu.sync_copy(abst_hbm.at[pl.ds(base, PER_W)], day_all)
    pltpu.sync_copy(dt_hbm.at[pl.ds(base, PER_W)], dt_all)

    halves = []
    for h in range(2):
        sl = pl.ds(h * L, L)
        wv, bv, twv = w_v[sl], b_v[sl], tw_v[sl]
        s = 1.0 / (1.0 + jnp.exp(-twv))
        halves.append((wv, bv, s, s - 1.0))

    outb_b = (outb0, outb1)
    se_b = (se0, se1)
    sw_b = (sw0, sw1)

    def gather_copy(c, d):
        lo = pl.multiple_of(c * CHUNK, CHUNK)
        return pltpu.make_async_copy(
            tab_hbm.at[idx_all.at[pl.ds(lo, CHUNK)]], outb_b[d], se_b[d])

    def write_copy(c, d):
        lo = pl.multiple_of(c * CHUNK, CHUNK)
        return pltpu.make_async_copy(
            outb_b[d],
            out_hbm.at[pl.ds(pl.multiple_of(base + lo, CHUNK), CHUNK)],
            sw_b[d])

    def compute_chunk(c, d):
        outb = outb_b[d]
        lo = pl.multiple_of(c * CHUNK, CHUNK)

        def grp_body(g, _):
            goff = pl.multiple_of(lo + g * L, L)
            dayv = lax.div(day_all[pl.ds(goff, L)], jnp.int32(24))
            dtv = dt_all[pl.ds(goff, L)]
            for l in range(L):
                p = g * L + l
                dayp = dayv[l]
                dtp = dtv[l]
                for h, (wv, bv, s, sm1) in enumerate(halves):
                    ncz = _neg_cos(wv * dtp + bv)
                    a = atab_v[dayp, pl.ds(h * L, L)]
                    outb[p, pl.ds(DIM_E + h * L, L)] = s * a + sm1 * ncz
            return 0

        lax.fori_loop(0, CHUNK // L, grp_body, 0)
        write_copy(c, d).start()

    gather_copy(0, 0).start()
    gather_copy(0, 0).wait()
    gather_copy(1, 1).start()
    compute_chunk(0, 0)

    def pair_body(k, _):
        c1 = 2 * k + 1
        gather_copy(c1, 1).wait()
        write_copy(c1 - 1, 0).wait()
        gather_copy(c1 + 1, 0).start()
        compute_chunk(c1, 1)

        c2 = 2 * k + 2
        gather_copy(c2, 0).wait()
        write_copy(c2 - 1, 1).wait()
        gather_copy(c2 + 1, 1).start()
        compute_chunk(c2, 0)
        return 0

    lax.fori_loop(0, (N_CHUNK - 2) // 2, pair_body, 0)

    gather_copy(N_CHUNK - 1, 1).wait()
    write_copy(N_CHUNK - 2, 0).wait()
    compute_chunk(N_CHUNK - 1, 1)
    write_copy(N_CHUNK - 1, 1).wait()


@jax.jit
def kernel(entities, dt, abst, ent_embs, w, b, t_w, abst_embs):
    ent_flat = entities.T.reshape(BS).astype(jnp.int32)
    abst_flat = abst.T.reshape(BS).astype(jnp.int32)
    dt_flat = dt.T.reshape(BS).astype(jnp.float32)

    tab_pad = _pad_table(ent_embs)

    mesh = plsc.VectorSubcoreMesh(core_axis_name="c", subcore_axis_name="s")
    run = pl.kernel(
        _sc_kernel,
        out_type=jax.ShapeDtypeStruct((BS, DIM_O), jnp.float32),
        mesh=mesh,
        scratch_types=[
            pltpu.VMEM((PER_W,), jnp.int32),
            pltpu.VMEM((PER_W,), jnp.int32),
            pltpu.VMEM((PER_W,), jnp.float32),
            pltpu.VMEM((N_ABST, DIM_T), jnp.float32),
            pltpu.VMEM((DIM_T,), jnp.float32),
            pltpu.VMEM((DIM_T,), jnp.float32),
            pltpu.VMEM((DIM_T,), jnp.float32),
            pltpu.VMEM((CHUNK, DIM_O), jnp.float32),
            pltpu.VMEM((CHUNK, DIM_O), jnp.float32),
            pltpu.SemaphoreType.DMA,
            pltpu.SemaphoreType.DMA,
            pltpu.SemaphoreType.DMA,
            pltpu.SemaphoreType.DMA,
        ],
    )
    out = run(ent_flat, abst_flat, dt_flat, tab_pad, w, b, t_w, abst_embs)
    return out.reshape(S, B, DIM_O).transpose(1, 0, 2)


# speedup vs baseline: 10.4073x; 1.1487x over previous
import jax
import jax.numpy as jnp
from jax import lax
from jax.experimental import pallas as pl
from jax.experimental.pallas import tpu as pltpu
from jax.experimental.pallas import tpu_sc as plsc

B, S = 4096, 50
BS = B * S
N_ENT, N_ABST = 100000, 365
DIM_E = 96
DIM_T = 32
DIM_O = 128

NC, NS, L = 2, 16, 16
NW = NC * NS
PER_W = BS // NW
CHUNK = 128
N_CHUNK = PER_W // CHUNK
NBUF = 3

_TWO_PI = 6.283185307179586
_PI = 3.141592653589793
_INV_TWO_PI = 0.15915494309189535
_C = (0.9999999848280942, -0.4999998592246148, 0.04166645028442878,
      -0.0013887630148866539, 2.4766406142774644e-05,
      -2.704700886591027e-07, 1.7134819119135614e-09)


def _neg_cos(x):
    k = (x * jnp.float32(_INV_TWO_PI)).astype(jnp.int32)
    y = x - k.astype(jnp.float32) * jnp.float32(_TWO_PI)
    y = jnp.where(y < 0.0, y + jnp.float32(_TWO_PI), y)
    z = y - jnp.float32(_PI)
    u = z * z
    p = jnp.full_like(u, _C[6])
    for i in (5, 4, 3, 2, 1, 0):
        p = p * u + jnp.float32(_C[i])
    return p


_PAD_BLK = 2048


def _pad_tc_kernel(src_ref, out_ref):
    x = src_ref[...]
    y = jnp.swapaxes(x, 0, 1)
    out_ref[...] = jnp.concatenate(
        [y, jnp.zeros((_PAD_BLK, DIM_O - DIM_E), jnp.float32)], axis=1)


def _pad_table(ent_embs):
    n_blk = (N_ENT + _PAD_BLK - 1) // _PAD_BLK
    return pl.pallas_call(
        _pad_tc_kernel,
        grid=(n_blk,),
        in_specs=[pl.BlockSpec((DIM_E, _PAD_BLK), lambda i: (0, i))],
        out_specs=pl.BlockSpec((_PAD_BLK, DIM_O), lambda i: (i, 0)),
        out_shape=jax.ShapeDtypeStruct((n_blk * _PAD_BLK, DIM_O), jnp.float32),
    )(ent_embs.T)


def _sc_kernel(ent_hbm, abst_hbm, dt_hbm, tab_hbm, w_hbm, b_hbm, tw_hbm,
               atab_hbm, out_hbm,
               idx_all, day_all, dt_all, atab_v, w_v, b_v, tw_v,
               outb0, outb1, outb2,
               se0, se1, se2, sw0, sw1, sw2):
    wid = lax.axis_index("s") * NC + lax.axis_index("c")
    base = pl.multiple_of(wid * PER_W, PER_W)

    pltpu.sync_copy(w_hbm, w_v)
    pltpu.sync_copy(b_hbm, b_v)
    pltpu.sync_copy(tw_hbm, tw_v)
    pltpu.sync_copy(atab_hbm, atab_v)
    pltpu.sync_copy(ent_hbm.at[pl.ds(base, PER_W)], idx_all)
    pltpu.sync_copy(abst_hbm.at[pl.ds(base, PER_W)], day_all)
    pltpu.sync_copy(dt_hbm.at[pl.ds(base, PER_W)], dt_all)

    halves = []
    for h in range(2):
        sl = pl.ds(h * L, L)
        wv, bv, twv = w_v[sl], b_v[sl], tw_v[sl]
        s = 1.0 / (1.0 + jnp.exp(-twv))
        halves.append((wv, bv, s, s - 1.0))

    outb_b = (outb0, outb1, outb2)
    se_b = (se0, se1, se2)
    sw_b = (sw0, sw1, sw2)

    def gather_copy(c, d):
        lo = pl.multiple_of(c * CHUNK, CHUNK)
        return pltpu.make_async_copy(
            tab_hbm.at[idx_all.at[pl.ds(lo, CHUNK)]], outb_b[d], se_b[d])

    def write_copy(c, d):
        lo = pl.multiple_of(c * CHUNK, CHUNK)
        return pltpu.make_async_copy(
            outb_b[d],
            out_hbm.at[pl.ds(pl.multiple_of(base + lo, CHUNK), CHUNK)],
            sw_b[d])

    def compute_chunk(c, d):
        outb = outb_b[d]
        lo = pl.multiple_of(c * CHUNK, CHUNK)

        def grp_body(g, _):
            goff = pl.multiple_of(lo + g * L, L)
            dayv = lax.div(day_all[pl.ds(goff, L)], jnp.int32(24))
            dtv = dt_all[pl.ds(goff, L)]
            for l in range(L):
                p = g * L + l
                dayp = dayv[l]
                dtp = dtv[l]
                for h, (wv, bv, s, sm1) in enumerate(halves):
                    ncz = _neg_cos(wv * dtp + bv)
                    a = atab_v[dayp, pl.ds(h * L, L)]
                    outb[p, pl.ds(DIM_E + h * L, L)] = s * a + sm1 * ncz
            return 0

        lax.fori_loop(0, CHUNK // L, grp_body, 0)
        write_copy(c, d).start()

    gather_copy(0, 0).start()
    gather_copy(0, 0).wait()
    gather_copy(1, 1).start()
    compute_chunk(0, 0)
    gather_copy(1, 1).wait()
    gather_copy(2, 2).start()
    compute_chunk(1, 1)

    def tri_body(k, _):
        for j in range(NBUF):
            c = NBUF * k + 2 + j
            r = (2 + j) % NBUF
            gather_copy(c, r).wait()
            write_copy(c - 2, (r + 1) % NBUF).wait()
            @pl.when(c + 1 < N_CHUNK)
            def _():
                gather_copy(c + 1, (r + 1) % NBUF).start()
            compute_chunk(c, r)
        return 0

    lax.fori_loop(0, (N_CHUNK - 2) // NBUF, tri_body, 0)

    write_copy(N_CHUNK - 2, (N_CHUNK - 2) % NBUF).wait()
    write_copy(N_CHUNK - 1, (N_CHUNK - 1) % NBUF).wait()


@jax.jit
def kernel(entities, dt, abst, ent_embs, w, b, t_w, abst_embs):
    ent_flat = entities.T.reshape(BS).astype(jnp.int32)
    abst_flat = abst.T.reshape(BS).astype(jnp.int32)
    dt_flat = dt.T.reshape(BS).astype(jnp.float32)

    tab_pad = _pad_table(ent_embs)

    mesh = plsc.VectorSubcoreMesh(core_axis_name="c", subcore_axis_name="s")
    run = pl.kernel(
        _sc_kernel,
        out_type=jax.ShapeDtypeStruct((BS, DIM_O), jnp.float32),
        mesh=mesh,
        scratch_types=[
            pltpu.VMEM((PER_W,), jnp.int32),
            pltpu.VMEM((PER_W,), jnp.int32),
            pltpu.VMEM((PER_W,), jnp.float32),
            pltpu.VMEM((N_ABST, DIM_T), jnp.float32),
            pltpu.VMEM((DIM_T,), jnp.float32),
            pltpu.VMEM((DIM_T,), jnp.float32),
            pltpu.VMEM((DIM_T,), jnp.float32),
            pltpu.VMEM((CHUNK, DIM_O), jnp.float32),
            pltpu.VMEM((CHUNK, DIM_O), jnp.float32),
            pltpu.VMEM((CHUNK, DIM_O), jnp.float32),
            pltpu.SemaphoreType.DMA,
            pltpu.SemaphoreType.DMA,
            pltpu.SemaphoreType.DMA,
            pltpu.SemaphoreType.DMA,
            pltpu.SemaphoreType.DMA,
            pltpu.SemaphoreType.DMA,
        ],
    )
    out = run(ent_flat, abst_flat, dt_flat, tab_pad, w, b, t_w, abst_embs)
    return out.reshape(S, B, DIM_O).transpose(1, 0, 2)
